# baseline (speedup 1.0000x reference)
import jax
import jax.numpy as jnp
from jax import lax
from jax.experimental import pallas as pl
from jax.experimental.pallas import tpu as pltpu
from jax.experimental.pallas import tpu_sc as plsc

N_DOCS = 1000000
B = 16384
L = 50
NC = 2
NS = 16
NW = NC * NS
ROWS = B // NW
NQ = 4
QR = ROWS // NQ
QN = QR * L


def _perm_body(x_hbm, idxt_hbm, idx_v, tbuf):
    wid = lax.axis_index("s") * NC + lax.axis_index("c")

    for q in range(NQ):
        col0 = wid * ROWS + q * QR
        pltpu.sync_copy(x_hbm.at[:, pl.ds(col0, QR)], idx_v)

        def perm(c, carry):
            b0 = c * 16
            for l in range(L):
                tbuf[pl.ds(l * QR + b0, 16)] = idx_v[l, pl.ds(b0, 16)]
            return carry

        lax.fori_loop(0, QR // 16, perm, 0)
        pltpu.sync_copy(tbuf, idxt_hbm.at[pl.ds(wid * (NQ * QN) + q * QN, QN)])


def _body(idxt_hbm, table_hbm, y_hbm, exam_hbm, rel_hbm,
          idxt0, idxt1, emb0, emb1,
          y0, ex0, rl0, y1, ex1, rl1, sg0, sg1, so0, so1):
    wid = lax.axis_index("s") * NC + lax.axis_index("c")

    idxt = [idxt0, idxt1]
    emb = [emb0, emb1]
    outs = [(y0, ex0, rl0), (y1, ex1, rl1)]
    sg = [sg0, sg1]
    so = [so0, so1]

    def load_permute(q):
        pltpu.sync_copy(idxt_hbm.at[pl.ds(wid * (NQ * QN) + q * QN, QN)],
                        idxt[q % 2])

    def start_gather(q):
        return pltpu.async_copy(table_hbm.at[idxt[q % 2]], emb[q % 2], sg[q % 2])

    gathers = [None] * NQ
    outcps = [None] * NQ

    load_permute(0)
    gathers[0] = start_gather(0)

    for q in range(NQ):
        nq = q + 1
        if nq < NQ:
            load_permute(nq)
            gathers[nq] = start_gather(nq)
        gathers[q].wait()
        if q >= 2:
            for cp in outcps[q - 2]:
                cp.wait()
        yb, eb, rb = outs[q % 2]
        ebuf = emb[q % 2]

        def casc(c, carry):
            b0 = c * 16
            ex = jnp.full((16,), 1.0, dtype=jnp.float32)
            for l in range(L):
                e = ebuf[pl.ds(l * QR + b0, 16)]
                r = 1.0 / (1.0 + jnp.exp(-e))
                rb[l, pl.ds(b0, 16)] = r
                eb[l, pl.ds(b0, 16)] = ex
                yb[l, pl.ds(b0, 16)] = ex * r
                ex = ex * (1.0 - r)
            return carry

        lax.fori_loop(0, QR // 16, casc, 0)

        col0 = wid * ROWS + q * QR
        sem = so[q % 2]
        outcps[q] = (
            pltpu.async_copy(yb, y_hbm.at[:, pl.ds(col0, QR)], sem),
            pltpu.async_copy(eb, exam_hbm.at[:, pl.ds(col0, QR)], sem),
            pltpu.async_copy(rb, rel_hbm.at[:, pl.ds(col0, QR)], sem),
        )

    for q in (NQ - 2, NQ - 1):
        for cp in outcps[q]:
            cp.wait()


def kernel(x, table):
    f32 = jnp.float32
    call = pl.kernel(
        _body,
        out_type=(
            jax.ShapeDtypeStruct((L, B), f32),
            jax.ShapeDtypeStruct((L, B), f32),
            jax.ShapeDtypeStruct((L, B), f32),
        ),
        mesh=plsc.VectorSubcoreMesh(core_axis_name="c", subcore_axis_name="s"),
        compiler_params=pltpu.CompilerParams(needs_layout_passes=False),
        scratch_types=[
            pltpu.VMEM((QN,), jnp.int32),
            pltpu.VMEM((QN,), jnp.int32),
            pltpu.VMEM((QN,), f32),
            pltpu.VMEM((QN,), f32),
            pltpu.VMEM((L, QR), f32),
            pltpu.VMEM((L, QR), f32),
            pltpu.VMEM((L, QR), f32),
            pltpu.VMEM((L, QR), f32),
            pltpu.VMEM((L, QR), f32),
            pltpu.VMEM((L, QR), f32),
            pltpu.SemaphoreType.DMA,
            pltpu.SemaphoreType.DMA,
            pltpu.SemaphoreType.DMA,
            pltpu.SemaphoreType.DMA,
        ],
    )
    perm = pl.kernel(
        _perm_body,
        out_type=jax.ShapeDtypeStruct((B * L,), jnp.int32),
        mesh=plsc.VectorSubcoreMesh(core_axis_name="c", subcore_axis_name="s"),
        compiler_params=pltpu.CompilerParams(needs_layout_passes=False),
        scratch_types=[
            pltpu.VMEM((L, QR), jnp.int32),
            pltpu.VMEM((QN,), jnp.int32),
        ],
    )
    t_flat = jnp.reshape(table, (N_DOCS,))
    xt = jnp.swapaxes(x.astype(jnp.int32), 0, 1)
    idxt_all = perm(xt)
    y, exam, rel = call(idxt_all, t_flat)
    return (
        jnp.swapaxes(y, 0, 1),
        jnp.swapaxes(exam, 0, 1),
        jnp.swapaxes(rel, 0, 1),
    )


# speedup vs baseline: 1.0201x; 1.0201x over previous
import jax
import jax.numpy as jnp
from jax import lax
from jax.experimental import pallas as pl
from jax.experimental.pallas import tpu as pltpu
from jax.experimental.pallas import tpu_sc as plsc

N_DOCS = 1000000
B = 16384
L = 50
NC = 2
NS = 16
NW = NC * NS
ROWS = B // NW
NQ = 4
QR = ROWS // NQ
QN = QR * L


def _body(x_hbm, table_hbm, y_hbm, exam_hbm, rel_hbm,
          idx_v, idxt0, idxt1, idxt2, idxt3, emb0, emb1, emb2, emb3,
          y0, ex0, rl0, y1, ex1, rl1, sg0, sg1, sg2, sg3, so0, so1):
    wid = lax.axis_index("s") * NC + lax.axis_index("c")

    idxt = [idxt0, idxt1, idxt2, idxt3]
    emb = [emb0, emb1, emb2, emb3]
    outs = [(y0, ex0, rl0), (y1, ex1, rl1)]
    sg = [sg0, sg1, sg2, sg3]
    so = [so0, so1]

    def load_permute(q):
        tbuf = idxt[q]
        col0 = wid * ROWS + q * QR
        pltpu.sync_copy(x_hbm.at[:, pl.ds(col0, QR)], idx_v)

        def perm(c, carry):
            b0 = c * 16
            for l in range(L):
                tbuf[pl.ds(l * QR + b0, 16)] = idx_v[l, pl.ds(b0, 16)]
            return carry

        lax.fori_loop(0, QR // 16, perm, 0)

    def start_gather(q):
        return pltpu.async_copy(table_hbm.at[idxt[q]], emb[q], sg[q])

    gathers = [None] * NQ
    outcps = [None] * NQ

    for q in range(NQ):
        load_permute(q)
        gathers[q] = start_gather(q)

    for q in range(NQ):
        gathers[q].wait()
        if q >= 2:
            for cp in outcps[q - 2]:
                cp.wait()
        yb, eb, rb = outs[q % 2]
        ebuf = emb[q]

        def casc(c, carry):
            b0 = c * 16
            ex = jnp.full((16,), 1.0, dtype=jnp.float32)
            for l in range(L):
                e = ebuf[pl.ds(l * QR + b0, 16)]
                r = 1.0 / (1.0 + jnp.exp(-e))
                rb[l, pl.ds(b0, 16)] = r
                eb[l, pl.ds(b0, 16)] = ex
                yb[l, pl.ds(b0, 16)] = ex * r
                ex = ex * (1.0 - r)
            return carry

        lax.fori_loop(0, QR // 16, casc, 0)

        col0 = wid * ROWS + q * QR
        sem = so[q % 2]
        outcps[q] = (
            pltpu.async_copy(yb, y_hbm.at[:, pl.ds(col0, QR)], sem),
            pltpu.async_copy(eb, exam_hbm.at[:, pl.ds(col0, QR)], sem),
            pltpu.async_copy(rb, rel_hbm.at[:, pl.ds(col0, QR)], sem),
        )

    for q in (NQ - 2, NQ - 1):
        for cp in outcps[q]:
            cp.wait()


def kernel(x, table):
    f32 = jnp.float32
    call = pl.kernel(
        _body,
        out_type=(
            jax.ShapeDtypeStruct((L, B), f32),
            jax.ShapeDtypeStruct((L, B), f32),
            jax.ShapeDtypeStruct((L, B), f32),
        ),
        mesh=plsc.VectorSubcoreMesh(core_axis_name="c", subcore_axis_name="s"),
        compiler_params=pltpu.CompilerParams(needs_layout_passes=False),
        scratch_types=[
            pltpu.VMEM((L, QR), jnp.int32),
            pltpu.VMEM((QN,), jnp.int32),
            pltpu.VMEM((QN,), jnp.int32),
            pltpu.VMEM((QN,), jnp.int32),
            pltpu.VMEM((QN,), jnp.int32),
            pltpu.VMEM((QN,), f32),
            pltpu.VMEM((QN,), f32),
            pltpu.VMEM((QN,), f32),
            pltpu.VMEM((QN,), f32),
            pltpu.VMEM((L, QR), f32),
            pltpu.VMEM((L, QR), f32),
            pltpu.VMEM((L, QR), f32),
            pltpu.VMEM((L, QR), f32),
            pltpu.VMEM((L, QR), f32),
            pltpu.VMEM((L, QR), f32),
            pltpu.SemaphoreType.DMA,
            pltpu.SemaphoreType.DMA,
            pltpu.SemaphoreType.DMA,
            pltpu.SemaphoreType.DMA,
            pltpu.SemaphoreType.DMA,
            pltpu.SemaphoreType.DMA,
        ],
    )
    t_flat = jnp.reshape(table, (N_DOCS,))
    xt = jnp.swapaxes(x.astype(jnp.int32), 0, 1)
    y, exam, rel = call(xt, t_flat)
    return (
        jnp.swapaxes(y, 0, 1),
        jnp.swapaxes(exam, 0, 1),
        jnp.swapaxes(rel, 0, 1),
    )
